# baseline (speedup 1.0000x reference)
import functools

import jax
import jax.numpy as jnp
from jax import lax
from jax.experimental import pallas as pl
from jax.experimental.pallas import tpu as pltpu
from jax.experimental.pallas import tpu_sc as plsc

_NUM_CORES = 2
_NUM_SUBCORES = 16
_NW = _NUM_CORES * _NUM_SUBCORES
_L = 16
_HC = 2


@functools.lru_cache(maxsize=None)
def _make_gather(BQ: int, H: int, D: int):
    assert BQ % _NW == 0
    bpt = BQ // _NW
    assert bpt % 128 == 0 and D % _L == 0 and H % _HC == 0
    npt = bpt * H
    nchunk = H // _HC
    rows = _HC * bpt
    assert nchunk % 2 == 0 and nchunk >= 6
    mesh = plsc.VectorSubcoreMesh(core_axis_name="c", subcore_axis_name="s")

    @functools.partial(
        pl.kernel,
        mesh=mesh,
        out_type=jax.ShapeDtypeStruct((H, D, BQ), jnp.float32),
        scratch_types=[
            pltpu.VMEM((npt,), jnp.int32),
            [pltpu.VMEM((rows,), jnp.int32) for _ in range(2)],
            [pltpu.VMEM((rows, 2 * D), jnp.float32) for _ in range(2)],
            [pltpu.VMEM((_HC, D, bpt), jnp.float32) for _ in range(2)],
            pltpu.SemaphoreType.DMA,
            pltpu.SemaphoreType.DMA,
        ],
        compiler_params=pltpu.CompilerParams(use_tc_tiling_on_sc=True,
                                             needs_layout_passes=False),
    )
    def gather(idx_hbm, table_hbm, out_hbm, idx_v, cidx, bufa, bufb,
               gsem, wsem):
        wid = lax.axis_index("s") * _NUM_CORES + lax.axis_index("c")
        b0 = wid * bpt
        pltpu.sync_copy(idx_hbm.at[pl.ds(b0 * H, npt)], idx_v)

        iota = jnp.arange(_L, dtype=jnp.int32)
        posv = [(k * _L + iota) * H for k in range(bpt // _L)]
        rowv = [[hh * bpt + k * _L + iota for k in range(bpt // _L)]
                for hh in range(_HC)]

        def build_cidx(c, ci):
            h0 = c * _HC
            for hh in range(_HC):
                for k in range(bpt // _L):
                    v = plsc.load_gather(idx_v, [posv[k] + (h0 + hh)])
                    ci.at[pl.ds(hh * bpt + k * _L, _L)][...] = v

        def g_start(ci, a):
            pltpu.async_copy(table_hbm.at[ci], a, gsem)

        def g_wait(ci, a):
            pltpu.make_async_copy(table_hbm.at[ci], a, gsem).wait()

        def w_start(c, b):
            pltpu.async_copy(
                b, out_hbm.at[pl.ds(c * _HC, _HC), :, pl.ds(b0, bpt)], wsem)

        def w_wait(b):
            pltpu.make_async_copy(
                b, out_hbm.at[pl.ds(0, _HC), :, pl.ds(b0, bpt)], wsem).wait()

        dvecs = [iota + _L * k for k in range(D // _L)]
        zeros = jnp.zeros((_L,), jnp.int32)

        def transpose(a, b):
            for hh in range(_HC):
                hv = zeros + hh

                def per_b(i, carry):
                    bp = i * 4
                    for j in range(4):
                        bv = zeros + (bp + j)
                        for k in range(D // _L):
                            v = a.at[hh * bpt + bp + j,
                                     pl.ds(k * _L, _L)][...]
                            plsc.store_scatter(b, [hv, dvecs[k], bv], v)
                    return carry
                lax.fori_loop(0, bpt // 4, per_b, 0)

        for p in range(2):
            build_cidx(p, cidx[p])
            g_start(cidx[p], bufa[p])
        for p in range(2):
            g_wait(cidx[p], bufa[p])
            transpose(bufa[p], bufb[p])
            build_cidx(2 + p, cidx[p])
            g_start(cidx[p], bufa[p])
            w_start(p, bufb[p])

        def body(j, carry):
            for p in range(2):
                c = 2 * j + p
                g_wait(cidx[p], bufa[p])
                w_wait(bufb[p])
                transpose(bufa[p], bufb[p])
                build_cidx(c + 2, cidx[p])
                g_start(cidx[p], bufa[p])
                w_start(c, bufb[p])
            return carry

        lax.fori_loop(1, nchunk // 2 - 1, body, 0)

        for p in range(2):
            c = nchunk - 2 + p
            g_wait(cidx[p], bufa[p])
            w_wait(bufb[p])
            transpose(bufa[p], bufb[p])
            w_start(c, bufb[p])
        for p in range(2):
            w_wait(bufb[p])

    return gather


def kernel(x, table):
    bq, hist = x.shape
    d = table.shape[1]
    idx = x.reshape(bq * hist).astype(jnp.int32)
    table_pad = jnp.pad(table, ((0, 0), (0, d)))
    out = _make_gather(bq, hist, d)(idx, table_pad)
    return jnp.transpose(out, (2, 0, 1))


# speedup vs baseline: 1.6661x; 1.6661x over previous
import functools

import jax
import jax.numpy as jnp
from jax import lax
from jax.experimental import pallas as pl
from jax.experimental.pallas import tpu as pltpu
from jax.experimental.pallas import tpu_sc as plsc

_NUM_CORES = 2
_NUM_SUBCORES = 16
_NW = _NUM_CORES * _NUM_SUBCORES
_LANES = 16
_ROWS_PER_STEP = 8


@functools.lru_cache(maxsize=None)
def _make_gather(BQ: int, H: int, D: int):
    assert BQ % _NW == 0
    rows_pw = BQ // _NW
    bpw = rows_pw * H
    assert rows_pw % 2 == 0 and rows_pw >= 4
    assert H % _ROWS_PER_STEP == 0 and D % _LANES == 0
    mesh = plsc.VectorSubcoreMesh(core_axis_name="c", subcore_axis_name="s")

    @functools.partial(
        pl.kernel,
        mesh=mesh,
        out_type=jax.ShapeDtypeStruct((BQ, H, D), jnp.float32),
        scratch_types=[
            pltpu.VMEM((bpw,), jnp.int32),
            [pltpu.VMEM((H, 2 * D), jnp.float32) for _ in range(2)],
            [pltpu.VMEM((H, D), jnp.float32) for _ in range(2)],
            pltpu.SemaphoreType.DMA,
            pltpu.SemaphoreType.DMA,
        ],
        compiler_params=pltpu.CompilerParams(use_tc_tiling_on_sc=True),
    )
    def gather(idx_hbm, table_hbm, out_hbm, idx_v, bufa, bufb, gsem, wsem):
        wid = lax.axis_index("s") * _NUM_CORES + lax.axis_index("c")
        base = wid * bpw
        row0 = wid * rows_pw
        pltpu.sync_copy(idx_hbm.at[pl.ds(base, bpw)], idx_v)

        def g_start(c, a):
            pltpu.async_copy(table_hbm.at[idx_v.at[pl.ds(c * H, H)]],
                             a, gsem)

        def g_wait(a):
            pltpu.make_async_copy(table_hbm.at[idx_v.at[pl.ds(0, H)]],
                                  a, gsem).wait()

        def w_start(c, b):
            pltpu.async_copy(b, out_hbm.at[row0 + c], wsem)

        def w_wait(b):
            pltpu.make_async_copy(b, out_hbm.at[row0], wsem).wait()

        def vcopy(a, b):
            def rows(i, carry):
                r = i * _ROWS_PER_STEP
                for j in range(_ROWS_PER_STEP):
                    for k in range(D // _LANES):
                        b.at[r + j, pl.ds(k * _LANES, _LANES)][...] = (
                            a.at[r + j, pl.ds(k * _LANES, _LANES)][...])
                return carry
            lax.fori_loop(0, H // _ROWS_PER_STEP, rows, 0)

        g_start(0, bufa[0])
        g_start(1, bufa[1])
        for p in range(2):
            g_wait(bufa[p])
            vcopy(bufa[p], bufb[p])
            g_start(2 + p, bufa[p])
            w_start(p, bufb[p])

        def body(j, carry):
            for p in range(2):
                c = 2 * j + p
                g_wait(bufa[p])
                w_wait(bufb[p])
                vcopy(bufa[p], bufb[p])
                g_start(c + 2, bufa[p])
                w_start(c, bufb[p])
            return carry

        lax.fori_loop(1, rows_pw // 2 - 1, body, 0)

        for p in range(2):
            c = rows_pw - 2 + p
            g_wait(bufa[p])
            w_wait(bufb[p])
            vcopy(bufa[p], bufb[p])
            w_start(c, bufb[p])
        for p in range(2):
            w_wait(bufb[p])

    return gather


def kernel(x, table):
    bq, hist = x.shape
    d = table.shape[1]
    idx = x.reshape(bq * hist).astype(jnp.int32)
    table_pad = jnp.pad(table, ((0, 0), (0, d)))
    return _make_gather(bq, hist, d)(idx, table_pad)


# speedup vs baseline: 1.6669x; 1.0005x over previous
import functools

import jax
import jax.numpy as jnp
from jax import lax
from jax.experimental import pallas as pl
from jax.experimental.pallas import tpu as pltpu
from jax.experimental.pallas import tpu_sc as plsc

_NUM_CORES = 2
_NUM_SUBCORES = 16
_NW = _NUM_CORES * _NUM_SUBCORES
_LANES = 16
_ROWS_PER_STEP = 8


@functools.lru_cache(maxsize=None)
def _make_gather(BQ: int, H: int, D: int):
    assert BQ % _NW == 0
    rows_pw = BQ // _NW
    bpw = rows_pw * H
    assert rows_pw % 2 == 0 and rows_pw >= 4
    assert H % _ROWS_PER_STEP == 0 and D % _LANES == 0
    mesh = plsc.VectorSubcoreMesh(core_axis_name="c", subcore_axis_name="s")

    @functools.partial(
        pl.kernel,
        mesh=mesh,
        out_type=jax.ShapeDtypeStruct((BQ, H, D), jnp.float32),
        scratch_types=[
            [pltpu.VMEM((H,), jnp.int32) for _ in range(3)],
            [pltpu.VMEM((H, 2 * D), jnp.float32) for _ in range(3)],
            [pltpu.VMEM((H, D), jnp.float32) for _ in range(2)],
            pltpu.SemaphoreType.DMA,
            pltpu.SemaphoreType.DMA,
            pltpu.SemaphoreType.DMA,
        ],
        compiler_params=pltpu.CompilerParams(use_tc_tiling_on_sc=True),
    )
    def gather(idx_hbm, table_hbm, out_hbm, cidx, bufa, bufb,
               isem, gsem, wsem):
        wid = lax.axis_index("s") * _NUM_CORES + lax.axis_index("c")
        base = wid * bpw
        row0 = wid * rows_pw

        def i_start(c, ci):
            pltpu.async_copy(idx_hbm.at[pl.ds(base + c * H, H)], ci, isem)

        def i_wait():
            pltpu.make_async_copy(idx_hbm.at[pl.ds(base, H)], cidx[0],
                                  isem).wait()

        def g_start(ci, a):
            pltpu.async_copy(table_hbm.at[ci], a, gsem)

        def g_wait(a):
            pltpu.make_async_copy(table_hbm.at[cidx[0]], a, gsem).wait()

        def w_start(c, b):
            pltpu.async_copy(b, out_hbm.at[row0 + c], wsem)

        def w_wait(b):
            pltpu.make_async_copy(b, out_hbm.at[row0], wsem).wait()

        def vcopy(a, b):
            def rows(i, carry):
                r = i * _ROWS_PER_STEP
                for j in range(_ROWS_PER_STEP):
                    for k in range(D // _LANES):
                        b.at[r + j, pl.ds(k * _LANES, _LANES)][...] = (
                            a.at[r + j, pl.ds(k * _LANES, _LANES)][...])
                return carry
            lax.fori_loop(0, H // _ROWS_PER_STEP, rows, 0)

        def step(c, pa, pb, first=False, last=False, prefetch=True):
            g_wait(bufa[pa])
            if not last:
                i_wait()
                g_start(cidx[(pa + 2) % 3], bufa[(pa + 2) % 3])
                if prefetch:
                    i_start(c + 3, cidx[pa])
            if not first:
                w_wait(bufb[pb])
            vcopy(bufa[pa], bufb[pb])
            w_start(c, bufb[pb])

        for p in range(3):
            i_start(p, cidx[p])
        for p in range(2):
            i_wait()
            g_start(cidx[p], bufa[p])
        step(0, 0, 0, first=True)
        step(1, 1, 1, first=True)

        def body(j, carry):
            for k in range(6):
                c = 2 + 6 * j + k
                step(c, (2 + k) % 3, k % 2)
            return carry

        lax.fori_loop(0, (rows_pw - 6) // 6, body, 0)

        for k in range(4):
            c = rows_pw - 6 + k
            step(c, (c % 3), c % 2, prefetch=(c + 3 < rows_pw))
        for k in range(2):
            c = rows_pw - 2 + k
            step(c, c % 3, c % 2, last=True)
        for p in range(2):
            w_wait(bufb[p])

    return gather


def kernel(x, table):
    bq, hist = x.shape
    d = table.shape[1]
    idx = x.reshape(bq * hist).astype(jnp.int32)
    table_pad = jnp.pad(table, ((0, 0), (0, d)))
    return _make_gather(bq, hist, d)(idx, table_pad)


# speedup vs baseline: 1.6731x; 1.0038x over previous
import functools

import jax
import jax.numpy as jnp
from jax import lax
from jax.experimental import pallas as pl
from jax.experimental.pallas import tpu as pltpu
from jax.experimental.pallas import tpu_sc as plsc

_NUM_CORES = 2
_NUM_SUBCORES = 16
_NW = _NUM_CORES * _NUM_SUBCORES
_LANES = 16
_ROWS_PER_STEP = 8


@functools.lru_cache(maxsize=None)
def _make_gather(BQ: int, H: int, D: int):
    assert BQ % _NW == 0
    rows_pw = BQ // _NW
    bpw = rows_pw * H
    assert rows_pw % 2 == 0 and rows_pw >= 4
    assert H % _ROWS_PER_STEP == 0 and D % _LANES == 0
    mesh = plsc.VectorSubcoreMesh(core_axis_name="c", subcore_axis_name="s")

    @functools.partial(
        pl.kernel,
        mesh=mesh,
        out_type=jax.ShapeDtypeStruct((BQ, H, D), jnp.float32),
        scratch_types=[
            pltpu.VMEM((bpw,), jnp.int32),
            [pltpu.VMEM((H, 2 * D), jnp.float32) for _ in range(2)],
            [pltpu.VMEM((H, D), jnp.float32) for _ in range(2)],
            pltpu.SemaphoreType.DMA,
            pltpu.SemaphoreType.DMA,
        ],
        compiler_params=pltpu.CompilerParams(use_tc_tiling_on_sc=True),
    )
    def gather(idx_hbm, table_hbm, out_hbm, idx_v, bufa, bufb, gsem, wsem):
        wid = lax.axis_index("s") * _NUM_CORES + lax.axis_index("c")
        base = wid * bpw
        row0 = wid * rows_pw
        pltpu.sync_copy(idx_hbm.at[pl.ds(base, bpw)], idx_v)

        def g_start(c, a):
            pltpu.async_copy(table_hbm.at[idx_v.at[pl.ds(c * H, H)]],
                             a, gsem)

        def g_wait(a):
            pltpu.make_async_copy(table_hbm.at[idx_v.at[pl.ds(0, H)]],
                                  a, gsem).wait()

        def w_start(c, b):
            pltpu.async_copy(b, out_hbm.at[row0 + c], wsem)

        def w_wait(b):
            pltpu.make_async_copy(b, out_hbm.at[row0], wsem).wait()

        def vcopy(a, b):
            def rows(i, carry):
                r = i * _ROWS_PER_STEP
                for j in range(_ROWS_PER_STEP):
                    for k in range(D // _LANES):
                        b.at[r + j, pl.ds(k * _LANES, _LANES)][...] = (
                            a.at[r + j, pl.ds(k * _LANES, _LANES)][...])
                return carry
            lax.fori_loop(0, H // _ROWS_PER_STEP, rows, 0)

        g_start(0, bufa[0])
        g_start(1, bufa[1])
        for p in range(2):
            g_wait(bufa[p])
            vcopy(bufa[p], bufb[p])
            g_start(2 + p, bufa[p])
            w_start(p, bufb[p])

        def body(j, carry):
            for p in range(2):
                c = 2 * j + p
                g_wait(bufa[p])
                w_wait(bufb[p])
                vcopy(bufa[p], bufb[p])
                g_start(c + 2, bufa[p])
                w_start(c, bufb[p])
            return carry

        lax.fori_loop(1, rows_pw // 2 - 1, body, 0)

        for p in range(2):
            c = rows_pw - 2 + p
            g_wait(bufa[p])
            w_wait(bufb[p])
            vcopy(bufa[p], bufb[p])
            w_start(c, bufb[p])
        for p in range(2):
            w_wait(bufb[p])

    return gather


def kernel(x, table):
    bq, hist = x.shape
    d = table.shape[1]
    idx = x.reshape(bq * hist).astype(jnp.int32)
    table_pad = jnp.pad(table, ((0, 0), (0, d)))
    return _make_gather(bq, hist, d)(idx, table_pad)
